# Initial kernel scaffold; baseline (speedup 1.0000x reference)
import functools

import jax
import jax.numpy as jnp
from jax import lax
from jax.experimental import pallas as pl
from jax.experimental.pallas import tpu as pltpu
from jax.experimental.pallas import tpu_sc as plsc

N = 10000
E = 320000
D_IN = 128
H1 = 128
H2 = 64
H3 = 64
G = 128

NC = 2
NS = 16
CHUNK = 128
CHUNKS = -(-E // (NC * NS * CHUNK))
E_PAD = NC * NS * CHUNKS * CHUNK

N_PAD = 10240
STRIPE = N_PAD // NS
DUMMY = 10200
R = 512
N_BLK = N_PAD // R

_MESH = plsc.VectorSubcoreMesh(core_axis_name="c", subcore_axis_name="s")


def _deg_body(dst_hbm, zeros_hbm, out_hbm, idx_v, ones_v, acc, sem):
    c = lax.axis_index("c")
    s = lax.axis_index("s")
    pltpu.sync_copy(dst_hbm.at[c, s], idx_v)

    def fill(j, carry):
        ones_v[j, :] = jnp.full((16,), 1.0, jnp.float32)
        return carry
    lax.fori_loop(0, CHUNK, fill, 0)

    pltpu.sync_copy(zeros_hbm.at[pl.ds(s * STRIPE, STRIPE)],
                    acc.at[pl.ds(s * STRIPE, STRIPE)])
    plsc.subcore_barrier()

    def step(j, carry):
        pltpu.sync_copy(ones_v, acc.at[idx_v.at[j]], add=True)
        return carry
    lax.fori_loop(0, CHUNKS, step, 0)

    plsc.subcore_barrier()
    pltpu.sync_copy(acc.at[pl.ds(s * STRIPE, STRIPE)],
                    out_hbm.at[c, pl.ds(s * STRIPE, STRIPE)])


_deg_kernel = functools.partial(
    pl.kernel,
    out_type=jax.ShapeDtypeStruct((NC, N_PAD, 16), jnp.float32),
    mesh=_MESH,
    scratch_types=[
        pltpu.VMEM((CHUNKS, CHUNK), jnp.int32),
        pltpu.VMEM((CHUNK, 16), jnp.float32),
        pltpu.VMEM_SHARED((N_PAD, 16), jnp.float32),
        pltpu.SemaphoreType.DMA,
    ],
)(_deg_body)


def _make_agg_kernel(h):
    def body(ht_hbm, src_hbm, dst_hbm, zeros_hbm, out_hbm,
             src_v, dst_v, rows_v, acc, sem):
        c = lax.axis_index("c")
        s = lax.axis_index("s")
        pltpu.sync_copy(src_hbm.at[c, s], src_v)
        pltpu.sync_copy(dst_hbm.at[c, s], dst_v)
        pltpu.sync_copy(zeros_hbm.at[pl.ds(s * STRIPE, STRIPE)],
                        acc.at[pl.ds(s * STRIPE, STRIPE)])
        plsc.subcore_barrier()

        def step(j, carry):
            pltpu.async_copy(ht_hbm.at[src_v.at[j]], rows_v, sem).wait()
            pltpu.sync_copy(rows_v, acc.at[dst_v.at[j]], add=True)
            return carry
        lax.fori_loop(0, CHUNKS, step, 0)

        plsc.subcore_barrier()
        pltpu.sync_copy(acc.at[pl.ds(s * STRIPE, STRIPE)],
                        out_hbm.at[c, pl.ds(s * STRIPE, STRIPE)])

    return functools.partial(
        pl.kernel,
        out_type=jax.ShapeDtypeStruct((NC, N_PAD, h), jnp.float32),
        mesh=_MESH,
        scratch_types=[
            pltpu.VMEM((CHUNKS, CHUNK), jnp.int32),
            pltpu.VMEM((CHUNKS, CHUNK), jnp.int32),
            pltpu.VMEM((CHUNK, h), jnp.float32),
            pltpu.VMEM_SHARED((N_PAD, h), jnp.float32),
            pltpu.SemaphoreType.DMA,
        ],
    )(body)


_agg128 = _make_agg_kernel(128)
_agg64 = _make_agg_kernel(64)


def _pre_body(x_ref, d0_ref, d1_ref, w_ref, ht_ref, dinv_ref):
    deg = d0_ref[...] + d1_ref[...] + 1.0
    dinv = lax.rsqrt(deg)
    h = jnp.dot(x_ref[...], w_ref[...], preferred_element_type=jnp.float32)
    ht_ref[...] = h * dinv
    dinv_ref[...] = dinv


def _pre_call(x, d0, d1, w):
    return pl.pallas_call(
        _pre_body,
        grid=(N_BLK,),
        in_specs=[
            pl.BlockSpec((R, D_IN), lambda i: (i, 0)),
            pl.BlockSpec((R, 1), lambda i: (i, 0)),
            pl.BlockSpec((R, 1), lambda i: (i, 0)),
            pl.BlockSpec((D_IN, H1), lambda i: (0, 0)),
        ],
        out_specs=[
            pl.BlockSpec((R, H1), lambda i: (i, 0)),
            pl.BlockSpec((R, 1), lambda i: (i, 0)),
        ],
        out_shape=[
            jax.ShapeDtypeStruct((N_PAD, H1), jnp.float32),
            jax.ShapeDtypeStruct((N_PAD, 1), jnp.float32),
        ],
    )(x, d0, d1, w)


def _mid_body(p0_ref, p1_ref, ht_ref, dinv_ref, b_ref, w_ref, out_ref):
    dinv = dinv_ref[...]
    raw = p0_ref[...] + p1_ref[...] + ht_ref[...]
    xl = jnp.maximum(raw * dinv + b_ref[...], 0.0)
    out_ref[...] = jnp.dot(
        xl, w_ref[...], preferred_element_type=jnp.float32) * dinv


def _mid_call(p0, p1, ht, dinv, b, w):
    h_in = ht.shape[1]
    h_out = w.shape[1]
    return pl.pallas_call(
        _mid_body,
        grid=(N_BLK,),
        in_specs=[
            pl.BlockSpec((R, h_in), lambda i: (i, 0)),
            pl.BlockSpec((R, h_in), lambda i: (i, 0)),
            pl.BlockSpec((R, h_in), lambda i: (i, 0)),
            pl.BlockSpec((R, 1), lambda i: (i, 0)),
            pl.BlockSpec((1, h_in), lambda i: (0, 0)),
            pl.BlockSpec((h_in, h_out), lambda i: (0, 0)),
        ],
        out_specs=pl.BlockSpec((R, h_out), lambda i: (i, 0)),
        out_shape=jax.ShapeDtypeStruct((N_PAD, h_out), jnp.float32),
    )(p0, p1, ht, dinv, b, w)


def _post_body(p0_ref, p1_ref, ht_ref, dinv_ref, b_ref, batch_ref, out_ref,
               sums_acc, cnt_acc):
    i = pl.program_id(0)

    @pl.when(i == 0)
    def _():
        sums_acc[...] = jnp.zeros_like(sums_acc)
        cnt_acc[...] = jnp.zeros_like(cnt_acc)

    dinv = dinv_ref[...]
    xl = jnp.maximum(
        (p0_ref[...] + p1_ref[...] + ht_ref[...]) * dinv + b_ref[...], 0.0)
    bt = batch_ref[0]
    gids = lax.broadcasted_iota(jnp.int32, (G, R), 0)
    mask = (jnp.broadcast_to(bt, (G, R)) == gids).astype(jnp.float32)
    sums_acc[...] += jnp.dot(mask, xl, preferred_element_type=jnp.float32)
    cnt_acc[...] += jnp.sum(mask, axis=1, keepdims=True)

    @pl.when(i == pl.num_programs(0) - 1)
    def _():
        out_ref[...] = sums_acc[...] / jnp.maximum(cnt_acc[...], 1.0)


def _post_call(p0, p1, ht, dinv, b, batch3d):
    h_in = ht.shape[1]
    return pl.pallas_call(
        _post_body,
        grid=(N_BLK,),
        in_specs=[
            pl.BlockSpec((R, h_in), lambda i: (i, 0)),
            pl.BlockSpec((R, h_in), lambda i: (i, 0)),
            pl.BlockSpec((R, h_in), lambda i: (i, 0)),
            pl.BlockSpec((R, 1), lambda i: (i, 0)),
            pl.BlockSpec((1, h_in), lambda i: (0, 0)),
            pl.BlockSpec((1, 1, R), lambda i: (i, 0, 0)),
        ],
        out_specs=pl.BlockSpec((G, h_in), lambda i: (0, 0)),
        out_shape=jax.ShapeDtypeStruct((G, h_in), jnp.float32),
        scratch_shapes=[
            pltpu.VMEM((G, h_in), jnp.float32),
            pltpu.VMEM((G, 1), jnp.float32),
        ],
    )(p0, p1, ht, dinv, b, batch3d)


def kernel(x, edge_index, batch, W1, b1, W2, b2, W3, b3):
    src = edge_index[0].astype(jnp.int32)
    dst = edge_index[1].astype(jnp.int32)
    src_r = jnp.pad(src, (0, E_PAD - E)).reshape(NC, NS, CHUNKS, CHUNK)
    dst_r = jnp.pad(dst, (0, E_PAD - E),
                    constant_values=DUMMY).reshape(NC, NS, CHUNKS, CHUNK)
    x_p = jnp.pad(x, ((0, N_PAD - N), (0, 0)))
    batch3d = jnp.pad(batch.astype(jnp.int32), (0, N_PAD - N),
                      constant_values=G).reshape(N_BLK, 1, R)
    zeros16 = jnp.zeros((N_PAD, 16), jnp.float32)
    zeros128 = jnp.zeros((N_PAD, 128), jnp.float32)
    zeros64 = jnp.zeros((N_PAD, 64), jnp.float32)

    degp = _deg_kernel(dst_r, zeros16)
    ht1, dinv = _pre_call(x_p, degp[0, :, :1], degp[1, :, :1], W1)
    p = _agg128(ht1, src_r, dst_r, zeros128)
    ht2 = _mid_call(p[0], p[1], ht1, dinv, b1.reshape(1, -1), W2)
    p = _agg64(ht2, src_r, dst_r, zeros64)
    ht3 = _mid_call(p[0], p[1], ht2, dinv, b2.reshape(1, -1), W3)
    p = _agg64(ht3, src_r, dst_r, zeros64)
    return _post_call(p[0], p[1], ht3, dinv, b3.reshape(1, -1), batch3d)


# speedup vs baseline: 13.0092x; 13.0092x over previous
import functools

import jax
import jax.numpy as jnp
from jax import lax
from jax.experimental import pallas as pl
from jax.experimental.pallas import tpu as pltpu
from jax.experimental.pallas import tpu_sc as plsc

N = 10000
E = 320000
D_IN = 128
H1 = 128
H2 = 64
H3 = 64
G = 128

NC = 2
NS = 16
CHUNK = 128
CHUNKS = -(-E // (NC * NS * CHUNK))
E_PAD = NC * NS * CHUNKS * CHUNK

N_PAD = 10240
STRIPE = N_PAD // NS
DUMMY = 10200
R = 512
N_BLK = N_PAD // R

_MESH = plsc.VectorSubcoreMesh(core_axis_name="c", subcore_axis_name="s",
                               num_cores=NC, num_subcores=NS)


def _deg_body(dst_hbm, zeros_hbm, out_hbm, idx_v, ones_v, acc, sem):
    c = lax.axis_index("c")
    s = lax.axis_index("s")
    pltpu.sync_copy(dst_hbm.at[c, s], idx_v)

    def fill(j, carry):
        ones_v[j, :] = jnp.full((16,), 1.0, jnp.float32)
        return carry
    lax.fori_loop(0, CHUNK, fill, 0)

    pltpu.sync_copy(zeros_hbm.at[pl.ds(s * STRIPE, STRIPE)],
                    acc.at[pl.ds(s * STRIPE, STRIPE)])
    plsc.subcore_barrier()

    def step(j, carry):
        pltpu.sync_copy(ones_v, acc.at[idx_v.at[j]], add=True)
        return carry
    lax.fori_loop(0, CHUNKS, step, 0)

    plsc.subcore_barrier()
    pltpu.sync_copy(acc.at[pl.ds(s * STRIPE, STRIPE)],
                    out_hbm.at[c, pl.ds(s * STRIPE, STRIPE)])


_deg_kernel = functools.partial(
    pl.kernel,
    out_type=jax.ShapeDtypeStruct((NC, N_PAD, 16), jnp.float32),
    mesh=_MESH,
    scratch_types=[
        pltpu.VMEM((CHUNKS, CHUNK), jnp.int32),
        pltpu.VMEM((CHUNK, 16), jnp.float32),
        pltpu.VMEM_SHARED((N_PAD, 16), jnp.float32),
        pltpu.SemaphoreType.DMA,
    ],
)(_deg_body)


def _make_agg_kernel(h):
    def body(ht_hbm, src_hbm, dst_hbm, zeros_hbm, out_hbm,
             src_v, dst_v, rows_v, acc, sem):
        c = lax.axis_index("c")
        s = lax.axis_index("s")
        pltpu.sync_copy(src_hbm.at[c, s], src_v)
        pltpu.sync_copy(dst_hbm.at[c, s], dst_v)
        pltpu.sync_copy(zeros_hbm.at[pl.ds(s * STRIPE, STRIPE)],
                        acc.at[pl.ds(s * STRIPE, STRIPE)])
        plsc.subcore_barrier()

        def step(j, carry):
            pltpu.async_copy(ht_hbm.at[src_v.at[j]], rows_v, sem).wait()
            pltpu.sync_copy(rows_v, acc.at[dst_v.at[j]], add=True)
            return carry
        lax.fori_loop(0, CHUNKS, step, 0)

        plsc.subcore_barrier()
        pltpu.sync_copy(acc.at[pl.ds(s * STRIPE, STRIPE)],
                        out_hbm.at[c, pl.ds(s * STRIPE, STRIPE)])

    return functools.partial(
        pl.kernel,
        out_type=jax.ShapeDtypeStruct((NC, N_PAD, h), jnp.float32),
        mesh=_MESH,
        compiler_params=pltpu.CompilerParams(use_tc_tiling_on_sc=False),
        scratch_types=[
            pltpu.VMEM((CHUNKS, CHUNK), jnp.int32),
            pltpu.VMEM((CHUNKS, CHUNK), jnp.int32),
            pltpu.VMEM((CHUNK, h), jnp.float32),
            pltpu.VMEM_SHARED((N_PAD, h), jnp.float32),
            pltpu.SemaphoreType.DMA,
        ],
    )(body)


_agg128 = _make_agg_kernel(128)
_agg64 = _make_agg_kernel(64)


def _pre_body(x_ref, d0_ref, d1_ref, w_ref, ht_ref, dinv_ref):
    deg = d0_ref[...] + d1_ref[...] + 1.0
    dinv = lax.rsqrt(deg)
    h = jnp.dot(x_ref[...], w_ref[...], preferred_element_type=jnp.float32)
    ht_ref[...] = h * dinv
    dinv_ref[...] = dinv


def _pre_call(x, d0, d1, w):
    return pl.pallas_call(
        _pre_body,
        grid=(N_BLK,),
        in_specs=[
            pl.BlockSpec((R, D_IN), lambda i: (i, 0)),
            pl.BlockSpec((R, 1), lambda i: (i, 0)),
            pl.BlockSpec((R, 1), lambda i: (i, 0)),
            pl.BlockSpec((D_IN, H1), lambda i: (0, 0)),
        ],
        out_specs=[
            pl.BlockSpec((R, H1), lambda i: (i, 0)),
            pl.BlockSpec((R, 1), lambda i: (i, 0)),
        ],
        out_shape=[
            jax.ShapeDtypeStruct((N_PAD, H1), jnp.float32),
            jax.ShapeDtypeStruct((N_PAD, 1), jnp.float32),
        ],
    )(x, d0, d1, w)


def _mid_body(p0_ref, p1_ref, ht_ref, dinv_ref, b_ref, w_ref, out_ref):
    dinv = dinv_ref[...]
    raw = p0_ref[...] + p1_ref[...] + ht_ref[...]
    xl = jnp.maximum(raw * dinv + b_ref[...], 0.0)
    out_ref[...] = jnp.dot(
        xl, w_ref[...], preferred_element_type=jnp.float32) * dinv


def _mid_call(p0, p1, ht, dinv, b, w):
    h_in = ht.shape[1]
    h_out = w.shape[1]
    return pl.pallas_call(
        _mid_body,
        grid=(N_BLK,),
        in_specs=[
            pl.BlockSpec((R, h_in), lambda i: (i, 0)),
            pl.BlockSpec((R, h_in), lambda i: (i, 0)),
            pl.BlockSpec((R, h_in), lambda i: (i, 0)),
            pl.BlockSpec((R, 1), lambda i: (i, 0)),
            pl.BlockSpec((1, h_in), lambda i: (0, 0)),
            pl.BlockSpec((h_in, h_out), lambda i: (0, 0)),
        ],
        out_specs=pl.BlockSpec((R, h_out), lambda i: (i, 0)),
        out_shape=jax.ShapeDtypeStruct((N_PAD, h_out), jnp.float32),
    )(p0, p1, ht, dinv, b, w)


def _post_body(p0_ref, p1_ref, ht_ref, dinv_ref, b_ref, batch_ref, out_ref,
               sums_acc, cnt_acc):
    i = pl.program_id(0)

    @pl.when(i == 0)
    def _():
        sums_acc[...] = jnp.zeros_like(sums_acc)
        cnt_acc[...] = jnp.zeros_like(cnt_acc)

    dinv = dinv_ref[...]
    xl = jnp.maximum(
        (p0_ref[...] + p1_ref[...] + ht_ref[...]) * dinv + b_ref[...], 0.0)
    bt = batch_ref[0]
    gids = lax.broadcasted_iota(jnp.int32, (G, R), 0)
    mask = (jnp.broadcast_to(bt, (G, R)) == gids).astype(jnp.float32)
    sums_acc[...] += jnp.dot(mask, xl, preferred_element_type=jnp.float32)
    cnt_acc[...] += jnp.sum(mask, axis=1, keepdims=True)

    @pl.when(i == pl.num_programs(0) - 1)
    def _():
        out_ref[...] = sums_acc[...] / jnp.maximum(cnt_acc[...], 1.0)


def _post_call(p0, p1, ht, dinv, b, batch3d):
    h_in = ht.shape[1]
    return pl.pallas_call(
        _post_body,
        grid=(N_BLK,),
        in_specs=[
            pl.BlockSpec((R, h_in), lambda i: (i, 0)),
            pl.BlockSpec((R, h_in), lambda i: (i, 0)),
            pl.BlockSpec((R, h_in), lambda i: (i, 0)),
            pl.BlockSpec((R, 1), lambda i: (i, 0)),
            pl.BlockSpec((1, h_in), lambda i: (0, 0)),
            pl.BlockSpec((1, 1, R), lambda i: (i, 0, 0)),
        ],
        out_specs=pl.BlockSpec((G, h_in), lambda i: (0, 0)),
        out_shape=jax.ShapeDtypeStruct((G, h_in), jnp.float32),
        scratch_shapes=[
            pltpu.VMEM((G, h_in), jnp.float32),
            pltpu.VMEM((G, 1), jnp.float32),
        ],
    )(p0, p1, ht, dinv, b, batch3d)


def kernel(x, edge_index, batch, W1, b1, W2, b2, W3, b3):
    src = edge_index[0].astype(jnp.int32)
    dst = edge_index[1].astype(jnp.int32)
    src_r = jnp.pad(src, (0, E_PAD - E)).reshape(NC, NS, CHUNKS, CHUNK)
    dst_r = jnp.pad(dst, (0, E_PAD - E),
                    constant_values=DUMMY).reshape(NC, NS, CHUNKS, CHUNK)
    x_p = jnp.pad(x, ((0, N_PAD - N), (0, 0)))
    batch3d = jnp.pad(batch.astype(jnp.int32), (0, N_PAD - N),
                      constant_values=G).reshape(N_BLK, 1, R)
    zeros16 = jnp.zeros((N_PAD, 16), jnp.float32)
    zeros128 = jnp.zeros((N_PAD, 128), jnp.float32)
    zeros64 = jnp.zeros((N_PAD, 64), jnp.float32)

    degp = _deg_kernel(dst_r, zeros16)
    ht1, dinv = _pre_call(x_p, degp[0, :, :1], degp[1, :, :1], W1)
    p = _agg128(ht1, src_r, dst_r, zeros128)
    ht2 = _mid_call(p[0], p[1], ht1, dinv, b1.reshape(1, -1), W2)
    p = _agg64(ht2, src_r, dst_r, zeros64)
    ht3 = _mid_call(p[0], p[1], ht2, dinv, b2.reshape(1, -1), W3)
    p = _agg64(ht3, src_r, dst_r, zeros64)
    return _post_call(p[0], p[1], ht3, dinv, b3.reshape(1, -1), batch3d)
